# baseline (speedup 1.0000x reference)
import functools

import jax
import jax.numpy as jnp
from jax import lax
from jax.experimental import pallas as pl
from jax.experimental.pallas import tpu as pltpu
from jax.experimental.pallas import tpu_sc as plsc

L = 16
NC = 2
NS = 16


def _ds8(off, size):
    return pl.ds(pl.multiple_of(off, 8), size)


def _sget(ref, k):
    idx = jnp.full((L,), k, jnp.int32)
    return jnp.max(plsc.load_gather(ref, [idx], mask=jnp.full((L,), True)))


def _make_sc_builder(N, E, B, D):
    NPAD = ((N + 127) // 128) * 128
    NSLOT = 2 * B
    EPT = E // NS
    CH = 4000
    assert EPT % CH == 0 and CH % L == 0
    CAP = 2 * E + 8 * NSLOT + 256
    TRASH = CAP - 8
    RELCAP = 2 * (E // NS) + 256
    PPT = B // (NC * NS)
    assert PPT * NC * NS == B and NSLOT % L == 0 and B % L == 0

    mesh = plsc.VectorSubcoreMesh(core_axis_name="c", subcore_axis_name="s",
                                  num_cores=NC, num_subcores=NS)

    @functools.partial(
        pl.kernel,
        out_type=jax.ShapeDtypeStruct((B * 4 * D,), jnp.float32),
        mesh=mesh,
        compiler_params=pltpu.CompilerParams(needs_layout_passes=False),
        scratch_types=dict(
            T_sh=pltpu.VMEM_SHARED((NPAD,), jnp.int32),
            H_sh=pltpu.VMEM_SHARED((NS, NSLOT), jnp.int32),
            LEN_sh=pltpu.VMEM_SHARED((NSLOT,), jnp.int32),
            BASE_sh=pltpu.VMEM_SHARED((NSLOT,), jnp.int32),
            CSR_sh=pltpu.VMEM_SHARED((CAP,), jnp.int32),
            REL_sh=pltpu.VMEM_SHARED((NS * RELCAP,), jnp.int32),
            T_loc=pltpu.VMEM((NPAD,), jnp.int32),
            cnt=pltpu.VMEM((NPAD,), jnp.int32),
            hist=pltpu.VMEM((NSLOT,), jnp.int32),
            cursor=pltpu.VMEM((NSLOT,), jnp.int32),
            hrow=pltpu.VMEM((NSLOT,), jnp.int32),
            base_loc=pltpu.VMEM((NSLOT,), jnp.int32),
            len_loc=pltpu.VMEM((NSLOT,), jnp.int32),
            ubuf=pltpu.VMEM((CH,), jnp.int32),
            vbuf=pltpu.VMEM((CH,), jnp.int32),
            tar_full=pltpu.VMEM((NSLOT,), jnp.int32),
            tari_loc=pltpu.VMEM((PPT,), jnp.int32),
            tarj_loc=pltpu.VMEM((PPT,), jnp.int32),
            slots_i=pltpu.VMEM((PPT,), jnp.int32),
            slots_j=pltpu.VMEM((PPT,), jnp.int32),
            lbuf=pltpu.VMEM((128,), jnp.int32),
            rbuf=pltpu.VMEM((128,), jnp.int32),
            stage_pos=pltpu.VMEM((160,), jnp.int32),
            stage_val=pltpu.VMEM((160,), jnp.int32),
            flush_pos=pltpu.VMEM((128,), jnp.int32),
            flush_val=pltpu.VMEM((128,), jnp.int32),
            stage_wv=pltpu.VMEM((L,), jnp.int32),
            stage_wc=pltpu.VMEM((L,), jnp.int32),
            xrow_i=pltpu.VMEM((D,), jnp.float32),
            xrow_j=pltpu.VMEM((D,), jnp.float32),
            xrow_w=pltpu.VMEM((D,), jnp.float32),
            xs_row=pltpu.VMEM((4 * D,), jnp.float32),
        ),
    )
    def sc_build(x_hbm, ei_hbm, tar_hbm, xs_hbm, *, T_sh, H_sh, LEN_sh,
                 BASE_sh, CSR_sh, T_loc, cnt, hist, cursor, hrow, base_loc,
                 len_loc, ubuf, vbuf, tar_full, tari_loc, tarj_loc, slots_i,
                 slots_j, lbuf, rbuf, REL_sh, stage_pos, stage_val, flush_pos,
                 flush_val, stage_wv, stage_wc, xrow_i, xrow_j, xrow_w,
                 xs_row):
        cid = lax.axis_index("c")
        sid = lax.axis_index("s")
        lane = lax.iota(jnp.int32, L)
        zeros16 = jnp.zeros((L,), jnp.int32)

        scope0 = jax.named_scope("sc_p0_slot_table")
        scope0.__enter__()
        @pl.when(sid == 0)
        def _():
            neg1 = jnp.full((L,), -1, jnp.int32)

            def t_init(i, _):
                T_loc[pl.ds(i * L, L)] = neg1
                return 0

            lax.fori_loop(0, NPAD // L, t_init, 0)
            pltpu.sync_copy(tar_hbm, tar_full)

            def t_scat(i, _):
                nodes = tar_full[pl.ds(i * L, L)]
                plsc.store_scatter(T_loc, [nodes], lane + i * L,
                                   mask=jnp.full((L,), True))
                return 0

            lax.fori_loop(0, NSLOT // L, t_scat, 0)
            pltpu.sync_copy(T_loc, T_sh)

        plsc.subcore_barrier()
        scope0.__exit__(None, None, None)

        scope1 = jax.named_scope("sc_p1_histogram")
        scope1.__enter__()
        pltpu.sync_copy(T_sh, T_loc)

        def z_hist(i, _):
            hist[pl.ds(i * L, L)] = zeros16
            return 0

        lax.fori_loop(0, NSLOT // L, z_hist, 0)

        def z_cnt(i, _):
            cnt[pl.ds(i * L, L)] = zeros16
            return 0

        lax.fori_loop(0, NPAD // L, z_cnt, 0)

        ebase = sid * EPT
        rbase = sid * RELCAP

        def hist_chunk(c, fill):
            pltpu.sync_copy(ei_hbm.at[_ds8(ebase + c * CH, CH)], ubuf)
            pltpu.sync_copy(ei_hbm.at[_ds8(E + ebase + c * CH, CH)], vbuf)

            def rel_flush(carry):
                fill, nfl = carry
                pltpu.sync_copy(stage_pos.at[pl.ds(0, 128)],
                                REL_sh.at[_ds8(rbase + nfl * 128, 128)])
                stage_pos[pl.ds(0, L)] = stage_pos[pl.ds(128, L)]
                return fill - 128, nfl + 1

            def hist_vec(i, carry):
                fill, nfl = carry
                for half in range(2):
                    u = ubuf[pl.ds(i * 2 * L + half * L, L)]
                    v = vbuf[pl.ds(i * 2 * L + half * L, L)]
                    for a, bb in ((u, v), (v, u)):
                        r = plsc.load_gather(T_loc, [a],
                                             mask=jnp.full((L,), True))
                        m = r >= 0
                        pk = (jnp.where(m, r, 0) << 14) | bb
                        plsc.store_compressed(stage_pos.at[pl.ds(fill, L)],
                                              pk, mask=m)
                        fill = fill + jnp.max(
                            plsc.all_reduce_population_count(m))
                        fill, nfl = lax.cond(fill >= 128, rel_flush,
                                             lambda c: c, (fill, nfl))
                return fill, nfl

            return lax.fori_loop(0, CH // (2 * L), hist_vec, fill)

        fill, nfl = lax.fori_loop(0, EPT // CH, hist_chunk,
                                  (jnp.int32(0), jnp.int32(0)))
        pltpu.sync_copy(stage_pos.at[pl.ds(0, 128)],
                        REL_sh.at[_ds8(rbase + nfl * 128, 128)])
        nrel = nfl * 128 + fill

        def hrel_chunk(cc, _):
            pltpu.sync_copy(REL_sh.at[_ds8(rbase + cc * 128, 128)], rbuf)

            def hrel_vec(q, _):
                pk = rbuf[pl.ds(q * L, L)]
                m = (cc * 128 + q * L + lane) < nrel
                rs = jnp.where(m, pk >> 14, 0)
                cntv, lastm = plsc.scan_count(rs, m)
                plsc.addupdate_scatter(hist, [rs], cntv,
                                       mask=jnp.logical_and(m, lastm))
                return 0

            lax.fori_loop(0, 8, hrel_vec, 0)
            return 0

        lax.fori_loop(0, (nrel + 127) // 128, hrel_chunk, 0)
        pltpu.sync_copy(hist, H_sh.at[sid])
        plsc.subcore_barrier()
        scope1.__exit__(None, None, None)

        scope2 = jax.named_scope("sc_p2_offsets")
        scope2.__enter__()
        @pl.when(sid == 0)
        def _():
            def z_len(i, _):
                len_loc[pl.ds(i * L, L)] = zeros16
                return 0

            lax.fori_loop(0, NSLOT // L, z_len, 0)

            def acc_tile(t, _):
                pltpu.sync_copy(H_sh.at[t], hrow)

                def acc_vec(i, _):
                    s = pl.ds(i * L, L)
                    len_loc[s] = len_loc[s] + hrow[s]
                    return 0

                lax.fori_loop(0, NSLOT // L, acc_vec, 0)
                return 0

            lax.fori_loop(0, NS, acc_tile, 0)
            pltpu.sync_copy(len_loc, LEN_sh)

            def base_vec(i, carry):
                s = pl.ds(i * L, L)
                lv = len_loc[s]
                lp = (lv + 7) & jnp.int32(~7)
                cs = plsc.cumsum(lp)
                base_loc[s] = carry + cs - lp
                return carry + jnp.max(cs)

            lax.fori_loop(0, NSLOT // L, base_vec, jnp.int32(0))
            pltpu.sync_copy(base_loc, BASE_sh)

        plsc.subcore_barrier()
        scope2.__exit__(None, None, None)

        scope3 = jax.named_scope("sc_p3_placement")
        scope3.__enter__()
        pltpu.sync_copy(BASE_sh, cursor)

        def pref_tile(t, _):
            pltpu.sync_copy(H_sh.at[t], hrow)

            def pref_vec(i, _):
                s = pl.ds(i * L, L)
                cursor[s] = cursor[s] + hrow[s]
                return 0

            lax.fori_loop(0, NSLOT // L, pref_vec, 0)
            return 0

        lax.fori_loop(0, sid, pref_tile, 0)

        def do_flush(f):
            for k in range(8):
                s = pl.ds(k * L, L)
                flush_pos[s] = stage_pos[s]
                flush_val[s] = stage_val[s]
            pltpu.sync_copy(flush_val, CSR_sh.at[flush_pos])
            stage_pos[pl.ds(0, L)] = stage_pos[pl.ds(128, L)]
            stage_val[pl.ds(0, L)] = stage_val[pl.ds(128, L)]
            return f - 128

        def place_chunk(cc, fill):
            pltpu.sync_copy(REL_sh.at[_ds8(rbase + cc * 128, 128)], rbuf)

            def place_vec(q, fill):
                pk = rbuf[pl.ds(q * L, L)]
                m = (cc * 128 + q * L + lane) < nrel
                rs = jnp.where(m, pk >> 14, 0)
                vv = pk & 16383
                cntv, lastm = plsc.scan_count(rs, m)
                before = plsc.load_gather(cursor, [rs], mask=m)
                pos = before + cntv - 1
                plsc.addupdate_scatter(cursor, [rs], cntv,
                                       mask=jnp.logical_and(m, lastm))
                plsc.store_compressed(stage_pos.at[pl.ds(fill, L)], pos,
                                      mask=m)
                plsc.store_compressed(stage_val.at[pl.ds(fill, L)], vv,
                                      mask=m)
                fill = fill + jnp.max(plsc.all_reduce_population_count(m))
                return lax.cond(fill >= 128, do_flush, lambda f: f, fill)

            return lax.fori_loop(0, 8, place_vec, fill)

        fill = lax.fori_loop(0, (nrel + 127) // 128, place_chunk,
                             jnp.int32(0))
        for k in range(8):
            s = pl.ds(k * L, L)
            g = lane + k * L
            flush_pos[s] = jnp.where(g < fill, stage_pos[s], TRASH)
            flush_val[s] = stage_val[s]
        pltpu.sync_copy(flush_val, CSR_sh.at[flush_pos])
        plsc.subcore_barrier()
        scope3.__exit__(None, None, None)

        scope4 = jax.named_scope("sc_p4_pairs")
        scope4.__enter__()
        pltpu.sync_copy(BASE_sh, base_loc)
        pltpu.sync_copy(LEN_sh, len_loc)
        pb = cid * (NS * PPT) + sid * PPT
        pltpu.sync_copy(tar_hbm.at[_ds8(pb, PPT)], tari_loc)
        pltpu.sync_copy(tar_hbm.at[_ds8(B + pb, PPT)], tarj_loc)

        for k in range(PPT // L):
            s = pl.ds(k * L, L)
            slots_i[s] = plsc.load_gather(T_loc, [tari_loc[s]],
                                          mask=jnp.full((L,), True))
            slots_j[s] = plsc.load_gather(T_loc, [tarj_loc[s]],
                                          mask=jnp.full((L,), True))

        zf16 = jnp.zeros((L,), jnp.float32)

        def pair_body(k, _):
            ib = _sget(tari_loc, k)
            jb = _sget(tarj_loc, k)
            ri = _sget(slots_i, k)
            rj = _sget(slots_j, k)
            baseA = _sget(base_loc, ri)
            lenA = _sget(len_loc, ri)
            baseB = _sget(base_loc, rj)
            lenB = _sget(len_loc, rj)

            for z in range(D // L):
                xs_row[pl.ds(3 * D + z * L, L)] = zf16

            def la_chunk(cc, _):
                pltpu.sync_copy(CSR_sh.at[_ds8(baseA + cc * 128, 128)], lbuf)

                def la_vec(q, _):
                    vv = lbuf[pl.ds(q * L, L)]
                    m = (cc * 128 + q * L + lane) < lenA
                    vs = jnp.where(m, vv, 0)
                    cntv, lastm = plsc.scan_count(vs, m)
                    plsc.addupdate_scatter(cnt, [vs], cntv,
                                           mask=jnp.logical_and(m, lastm))
                    return 0

                lax.fori_loop(0, 8, la_vec, 0)
                return 0

            lax.fori_loop(0, (lenA + 127) // 128, la_chunk, 0)

            w = _sget(cnt, jb).astype(jnp.float32)

            def lb_chunk(cc, _):
                pltpu.sync_copy(CSR_sh.at[_ds8(baseB + cc * 128, 128)], lbuf)

                def lb_vec(q, _):
                    vv = lbuf[pl.ds(q * L, L)]
                    m = (cc * 128 + q * L + lane) < lenB
                    vs = jnp.where(m, vv, 0)
                    cv = plsc.load_gather(cnt, [vs], mask=m)
                    wm = jnp.logical_and(m, cv > 0)
                    nn = jnp.max(plsc.all_reduce_population_count(wm))

                    @pl.when(nn > 0)
                    def _():
                        plsc.store_compressed(stage_wv.at[pl.ds(0, L)], vs,
                                              mask=wm)
                        plsc.store_compressed(stage_wc.at[pl.ds(0, L)], cv,
                                              mask=wm)

                        def work(t, _):
                            wv = _sget(stage_wv, t)
                            wc = _sget(stage_wc, t).astype(jnp.float32)
                            pltpu.sync_copy(x_hbm.at[_ds8(wv * D, D)],
                                            xrow_w)
                            for z in range(D // L):
                                sl = pl.ds(3 * D + z * L, L)
                                xs_row[sl] = (xs_row[sl]
                                              + wc * xrow_w[pl.ds(z * L, L)])
                            return 0

                        lax.fori_loop(0, nn, work, 0)

                    return 0

                lax.fori_loop(0, 8, lb_vec, 0)
                return 0

            lax.fori_loop(0, (lenB + 127) // 128, lb_chunk, 0)

            def lz_chunk(cc, _):
                pltpu.sync_copy(CSR_sh.at[_ds8(baseA + cc * 128, 128)], lbuf)

                def lz_vec(q, _):
                    vv = lbuf[pl.ds(q * L, L)]
                    m = (cc * 128 + q * L + lane) < lenA
                    vs = jnp.where(m, vv, 0)
                    plsc.store_scatter(cnt, [vs], zeros16, mask=m)
                    return 0

                lax.fori_loop(0, 8, lz_vec, 0)
                return 0

            lax.fori_loop(0, (lenA + 127) // 128, lz_chunk, 0)

            pltpu.sync_copy(x_hbm.at[_ds8(ib * D, D)], xrow_i)
            pltpu.sync_copy(x_hbm.at[_ds8(jb * D, D)], xrow_j)
            for z in range(D // L):
                sz = pl.ds(z * L, L)
                xi = xrow_i[sz]
                xj = xrow_j[sz]
                xs_row[pl.ds(z * L, L)] = xi * xj
                xs_row[pl.ds(D + z * L, L)] = w * xi
                xs_row[pl.ds(2 * D + z * L, L)] = w * xj
            pltpu.sync_copy(xs_row, xs_hbm.at[_ds8((pb + k) * 4 * D, 4 * D)])
            return 0

        lax.fori_loop(0, PPT, pair_body, 0)
        scope4.__exit__(None, None, None)

    return sc_build


def _mlp_body(xs_ref, wlin_ref, blin_ref, w1_ref, b1_ref, w2_ref, b2_ref,
              out_ref):
    xs = xs_ref[...]
    h = jnp.dot(xs, w1_ref[...], preferred_element_type=jnp.float32)
    h = jnp.maximum(h + b1_ref[...], 0.0)
    out = jnp.dot(xs, wlin_ref[...], preferred_element_type=jnp.float32)
    out = out + jnp.dot(h, w2_ref[...], preferred_element_type=jnp.float32)
    out_ref[...] = out + blin_ref[...] + b2_ref[...]


def kernel(x, edge_index, tar_ei, W_lin, b_lin, W1, b1, W2, b2):
    N, D = x.shape
    E = edge_index.shape[1]
    B = tar_ei.shape[1]
    sc_build = _make_sc_builder(N, E, B, D)
    xs = sc_build(x.reshape(-1), edge_index.astype(jnp.int32).reshape(-1),
                  tar_ei.astype(jnp.int32).reshape(-1)).reshape(B, 4 * D)
    out = pl.pallas_call(
        _mlp_body,
        out_shape=jax.ShapeDtypeStruct((B, W_lin.shape[1]), jnp.float32),
    )(xs, W_lin, b_lin.reshape(1, -1), W1, b1.reshape(1, -1), W2,
      b2.reshape(1, -1))
    return out


# speedup vs baseline: 1.2516x; 1.2516x over previous
import functools

import jax
import jax.numpy as jnp
from jax import lax
from jax.experimental import pallas as pl
from jax.experimental.pallas import tpu as pltpu
from jax.experimental.pallas import tpu_sc as plsc

L = 16
NC = 2
NS = 16


def _ds8(off, size):
    return pl.ds(pl.multiple_of(off, 8), size)


def _sget(ref, k):
    idx = jnp.full((L,), k, jnp.int32)
    return jnp.max(plsc.load_gather(ref, [idx], mask=jnp.full((L,), True)))


def _make_sc_builder(N, E, B, D):
    NPAD = ((N + 127) // 128) * 128
    NSLOT = 2 * B
    EPT = E // NS
    CH = 4000
    assert EPT % CH == 0 and CH % L == 0
    CAP = 2 * E + 8 * NSLOT + 256
    TRASH = CAP - 8
    RELCAP = 2 * (E // NS) + 256
    PPT = B // (NC * NS)
    assert PPT * NC * NS == B and NSLOT % L == 0 and B % L == 0

    mesh = plsc.VectorSubcoreMesh(core_axis_name="c", subcore_axis_name="s",
                                  num_cores=NC, num_subcores=NS)

    @functools.partial(
        pl.kernel,
        out_type=jax.ShapeDtypeStruct((B * 4 * D,), jnp.float32),
        mesh=mesh,
        compiler_params=pltpu.CompilerParams(needs_layout_passes=False),
        scratch_types=dict(
            T_sh=pltpu.VMEM_SHARED((NPAD,), jnp.int32),
            H_sh=pltpu.VMEM_SHARED((NS, NSLOT), jnp.int32),
            LEN_sh=pltpu.VMEM_SHARED((NSLOT,), jnp.int32),
            BASE_sh=pltpu.VMEM_SHARED((NSLOT,), jnp.int32),
            CSR_sh=pltpu.VMEM_SHARED((CAP,), jnp.int32),
            REL_sh=pltpu.VMEM_SHARED((NS * RELCAP,), jnp.int32),
            T_loc=pltpu.VMEM((NPAD,), jnp.int32),
            cnt=pltpu.VMEM((NPAD,), jnp.int32),
            hist=pltpu.VMEM((NSLOT,), jnp.int32),
            cursor=pltpu.VMEM((NSLOT,), jnp.int32),
            hrow=pltpu.VMEM((NSLOT,), jnp.int32),
            base_loc=pltpu.VMEM((NSLOT,), jnp.int32),
            len_loc=pltpu.VMEM((NSLOT,), jnp.int32),
            ubuf=pltpu.VMEM((CH,), jnp.int32),
            vbuf=pltpu.VMEM((CH,), jnp.int32),
            tar_full=pltpu.VMEM((NSLOT,), jnp.int32),
            tari_loc=pltpu.VMEM((PPT,), jnp.int32),
            tarj_loc=pltpu.VMEM((PPT,), jnp.int32),
            slots_i=pltpu.VMEM((PPT,), jnp.int32),
            slots_j=pltpu.VMEM((PPT,), jnp.int32),
            lbuf=pltpu.VMEM((128,), jnp.int32),
            rbuf=pltpu.VMEM((128,), jnp.int32),
            stage_pos=pltpu.VMEM((160,), jnp.int32),
            stage_val=pltpu.VMEM((160,), jnp.int32),
            flush_pos=pltpu.VMEM((128,), jnp.int32),
            flush_val=pltpu.VMEM((128,), jnp.int32),
            stage_wv=pltpu.VMEM((L,), jnp.int32),
            stage_wc=pltpu.VMEM((L,), jnp.int32),
            xrow_i=pltpu.VMEM((D,), jnp.float32),
            xrow_j=pltpu.VMEM((D,), jnp.float32),
            xrow_w=pltpu.VMEM((D,), jnp.float32),
            xs_row=pltpu.VMEM((4 * D,), jnp.float32),
            psem=pltpu.SemaphoreType.DMA,
        ),
    )
    def sc_build(x_hbm, ei_hbm, tar_hbm, xs_hbm, *, T_sh, H_sh, LEN_sh,
                 BASE_sh, CSR_sh, T_loc, cnt, hist, cursor, hrow, base_loc,
                 len_loc, ubuf, vbuf, tar_full, tari_loc, tarj_loc, slots_i,
                 slots_j, lbuf, rbuf, REL_sh, stage_pos, stage_val, flush_pos,
                 flush_val, stage_wv, stage_wc, xrow_i, xrow_j, xrow_w,
                 xs_row, psem):
        cid = lax.axis_index("c")
        sid = lax.axis_index("s")
        lane = lax.iota(jnp.int32, L)
        zeros16 = jnp.zeros((L,), jnp.int32)

        scope0 = jax.named_scope("sc_p0_slot_table")
        scope0.__enter__()
        @pl.when(sid == 0)
        def _():
            neg1 = jnp.full((L,), -1, jnp.int32)

            def t_init(i, _):
                T_loc[pl.ds(i * L, L)] = neg1
                return 0

            lax.fori_loop(0, NPAD // L, t_init, 0)
            pltpu.sync_copy(tar_hbm, tar_full)

            def t_scat(i, _):
                nodes = tar_full[pl.ds(i * L, L)]
                plsc.store_scatter(T_loc, [nodes], lane + i * L,
                                   mask=jnp.full((L,), True))
                return 0

            lax.fori_loop(0, NSLOT // L, t_scat, 0)
            pltpu.sync_copy(T_loc, T_sh)

        plsc.subcore_barrier()
        scope0.__exit__(None, None, None)

        scope1 = jax.named_scope("sc_p1_histogram")
        scope1.__enter__()
        pltpu.sync_copy(T_sh, T_loc)

        def z_hist(i, _):
            hist[pl.ds(i * L, L)] = zeros16
            return 0

        lax.fori_loop(0, NSLOT // L, z_hist, 0)

        def z_cnt(i, _):
            cnt[pl.ds(i * L, L)] = zeros16
            return 0

        lax.fori_loop(0, NPAD // L, z_cnt, 0)

        ebase = sid * EPT
        rbase = sid * RELCAP

        def hist_chunk(c, fill):
            pltpu.sync_copy(ei_hbm.at[_ds8(ebase + c * CH, CH)], ubuf)
            pltpu.sync_copy(ei_hbm.at[_ds8(E + ebase + c * CH, CH)], vbuf)

            def rel_flush(carry):
                fill, nfl = carry
                pltpu.sync_copy(stage_pos.at[pl.ds(0, 128)],
                                REL_sh.at[_ds8(rbase + nfl * 128, 128)])
                stage_pos[pl.ds(0, L)] = stage_pos[pl.ds(128, L)]
                return fill - 128, nfl + 1

            def hist_vec(i, carry):
                fill, nfl = carry
                u = ubuf[pl.ds(i * L, L)]
                v = vbuf[pl.ds(i * L, L)]
                for a, bb in ((u, v), (v, u)):
                    r = plsc.load_gather(T_loc, [a], mask=jnp.full((L,), True))
                    m = r >= 0
                    rs = jnp.where(m, r, 0)
                    cntv, lastm = plsc.scan_count(rs, m)
                    plsc.addupdate_scatter(hist, [rs], cntv,
                                           mask=jnp.logical_and(m, lastm))
                    pk = (rs << 14) | bb
                    plsc.store_compressed(stage_pos.at[pl.ds(fill, L)], pk,
                                          mask=m)
                    fill = fill + jnp.max(
                        plsc.all_reduce_population_count(m))
                    fill, nfl = lax.cond(fill >= 128, rel_flush,
                                         lambda c: c, (fill, nfl))
                return fill, nfl

            return lax.fori_loop(0, CH // L, hist_vec, fill)

        fill, nfl = lax.fori_loop(0, EPT // CH, hist_chunk,
                                  (jnp.int32(0), jnp.int32(0)))
        pltpu.sync_copy(stage_pos.at[pl.ds(0, 128)],
                        REL_sh.at[_ds8(rbase + nfl * 128, 128)])
        nrel = nfl * 128 + fill
        pltpu.sync_copy(hist, H_sh.at[sid])
        plsc.subcore_barrier()
        scope1.__exit__(None, None, None)

        scope2 = jax.named_scope("sc_p2_offsets")
        scope2.__enter__()
        @pl.when(sid == 0)
        def _():
            def z_len(i, _):
                len_loc[pl.ds(i * L, L)] = zeros16
                return 0

            lax.fori_loop(0, NSLOT // L, z_len, 0)

            def acc_tile(t, _):
                pltpu.sync_copy(H_sh.at[t], hrow)

                def acc_vec(i, _):
                    s = pl.ds(i * L, L)
                    len_loc[s] = len_loc[s] + hrow[s]
                    return 0

                lax.fori_loop(0, NSLOT // L, acc_vec, 0)
                return 0

            lax.fori_loop(0, NS, acc_tile, 0)
            pltpu.sync_copy(len_loc, LEN_sh)

            def base_vec(i, carry):
                s = pl.ds(i * L, L)
                lv = len_loc[s]
                lp = (lv + 7) & jnp.int32(~7)
                cs = plsc.cumsum(lp)
                base_loc[s] = carry + cs - lp
                return carry + jnp.max(cs)

            lax.fori_loop(0, NSLOT // L, base_vec, jnp.int32(0))
            pltpu.sync_copy(base_loc, BASE_sh)

        plsc.subcore_barrier()
        scope2.__exit__(None, None, None)

        scope3 = jax.named_scope("sc_p3_placement")
        scope3.__enter__()
        pltpu.sync_copy(BASE_sh, cursor)

        def pref_tile(t, _):
            pltpu.sync_copy(H_sh.at[t], hrow)

            def pref_vec(i, _):
                s = pl.ds(i * L, L)
                cursor[s] = cursor[s] + hrow[s]
                return 0

            lax.fori_loop(0, NSLOT // L, pref_vec, 0)
            return 0

        lax.fori_loop(0, sid, pref_tile, 0)

        def do_flush(f):
            for k in range(8):
                s = pl.ds(k * L, L)
                flush_pos[s] = stage_pos[s]
                flush_val[s] = stage_val[s]
            pltpu.sync_copy(flush_val, CSR_sh.at[flush_pos])
            stage_pos[pl.ds(0, L)] = stage_pos[pl.ds(128, L)]
            stage_val[pl.ds(0, L)] = stage_val[pl.ds(128, L)]
            return f - 128

        def place_chunk(cc, fill):
            pltpu.sync_copy(REL_sh.at[_ds8(rbase + cc * 128, 128)], rbuf)

            def place_vec(q, fill):
                pk = rbuf[pl.ds(q * L, L)]
                m = (cc * 128 + q * L + lane) < nrel
                rs = jnp.where(m, pk >> 14, 0)
                vv = pk & 16383
                cntv, lastm = plsc.scan_count(rs, m)
                before = plsc.load_gather(cursor, [rs], mask=m)
                pos = before + cntv - 1
                plsc.addupdate_scatter(cursor, [rs], cntv,
                                       mask=jnp.logical_and(m, lastm))
                plsc.store_compressed(stage_pos.at[pl.ds(fill, L)], pos,
                                      mask=m)
                plsc.store_compressed(stage_val.at[pl.ds(fill, L)], vv,
                                      mask=m)
                fill = fill + jnp.max(plsc.all_reduce_population_count(m))
                return lax.cond(fill >= 128, do_flush, lambda f: f, fill)

            return lax.fori_loop(0, 8, place_vec, fill)

        fill = lax.fori_loop(0, (nrel + 127) // 128, place_chunk,
                             jnp.int32(0))
        for k in range(8):
            s = pl.ds(k * L, L)
            g = lane + k * L
            flush_pos[s] = jnp.where(g < fill, stage_pos[s], TRASH)
            flush_val[s] = stage_val[s]
        pltpu.sync_copy(flush_val, CSR_sh.at[flush_pos])
        plsc.subcore_barrier()
        scope3.__exit__(None, None, None)

        scope4 = jax.named_scope("sc_p4_pairs")
        scope4.__enter__()
        pltpu.sync_copy(BASE_sh, base_loc)
        pltpu.sync_copy(LEN_sh, len_loc)
        pb = cid * (NS * PPT) + sid * PPT
        pltpu.sync_copy(tar_hbm.at[_ds8(pb, PPT)], tari_loc)
        pltpu.sync_copy(tar_hbm.at[_ds8(B + pb, PPT)], tarj_loc)

        for k in range(PPT // L):
            s = pl.ds(k * L, L)
            slots_i[s] = plsc.load_gather(T_loc, [tari_loc[s]],
                                          mask=jnp.full((L,), True))
            slots_j[s] = plsc.load_gather(T_loc, [tarj_loc[s]],
                                          mask=jnp.full((L,), True))

        zf16 = jnp.zeros((L,), jnp.float32)

        def pair_body(k, _):
            ib = _sget(tari_loc, k)
            jb = _sget(tarj_loc, k)
            ri = _sget(slots_i, k)
            rj = _sget(slots_j, k)
            baseA = _sget(base_loc, ri)
            lenA = _sget(len_loc, ri)
            baseB = _sget(base_loc, rj)
            lenB = _sget(len_loc, rj)
            ncA = (lenA + 127) // 128
            nvA = (lenA + L - 1) // L
            nvB = (lenB + L - 1) // L

            dxi = pltpu.async_copy(x_hbm.at[_ds8(ib * D, D)], xrow_i, psem)
            dxj = pltpu.async_copy(x_hbm.at[_ds8(jb * D, D)], xrow_j, psem)

            for z in range(D // L):
                xs_row[pl.ds(3 * D + z * L, L)] = zf16

            def la_chunk(cc, _):
                pltpu.sync_copy(CSR_sh.at[_ds8(baseA + cc * 128, 128)], lbuf)

                def la_vec(q, _):
                    vv = lbuf[pl.ds(q * L, L)]
                    m = (cc * 128 + q * L + lane) < lenA
                    vs = jnp.where(m, vv, 0)
                    cntv, lastm = plsc.scan_count(vs, m)
                    plsc.addupdate_scatter(cnt, [vs], cntv,
                                           mask=jnp.logical_and(m, lastm))
                    return 0

                lax.fori_loop(0, jnp.minimum(8, nvA - cc * 8), la_vec, 0)
                return 0

            lax.fori_loop(0, ncA, la_chunk, 0)

            w = _sget(cnt, jb).astype(jnp.float32)

            def lb_chunk(cc, _):
                pltpu.sync_copy(CSR_sh.at[_ds8(baseB + cc * 128, 128)], rbuf)

                def lb_vec(q, _):
                    vv = rbuf[pl.ds(q * L, L)]
                    m = (cc * 128 + q * L + lane) < lenB
                    vs = jnp.where(m, vv, 0)
                    cv = plsc.load_gather(cnt, [vs], mask=m)
                    wm = jnp.logical_and(m, cv > 0)
                    nn = jnp.max(plsc.all_reduce_population_count(wm))

                    @pl.when(nn > 0)
                    def _():
                        plsc.store_compressed(stage_wv.at[pl.ds(0, L)], vs,
                                              mask=wm)
                        plsc.store_compressed(stage_wc.at[pl.ds(0, L)], cv,
                                              mask=wm)

                        def work(t, _):
                            wv = _sget(stage_wv, t)
                            wc = _sget(stage_wc, t).astype(jnp.float32)
                            pltpu.sync_copy(x_hbm.at[_ds8(wv * D, D)],
                                            xrow_w)
                            for z in range(D // L):
                                sl = pl.ds(3 * D + z * L, L)
                                xs_row[sl] = (xs_row[sl]
                                              + wc * xrow_w[pl.ds(z * L, L)])
                            return 0

                        lax.fori_loop(0, nn, work, 0)

                    return 0

                lax.fori_loop(0, jnp.minimum(8, nvB - cc * 8), lb_vec, 0)
                return 0

            lax.fori_loop(0, (lenB + 127) // 128, lb_chunk, 0)

            def lz_chunk(cc, _):
                @pl.when(ncA > 1)
                def _():
                    pltpu.sync_copy(CSR_sh.at[_ds8(baseA + cc * 128, 128)],
                                    lbuf)

                def lz_vec(q, _):
                    vv = lbuf[pl.ds(q * L, L)]
                    m = (cc * 128 + q * L + lane) < lenA
                    vs = jnp.where(m, vv, 0)
                    plsc.store_scatter(cnt, [vs], zeros16, mask=m)
                    return 0

                lax.fori_loop(0, jnp.minimum(8, nvA - cc * 8), lz_vec, 0)
                return 0

            lax.fori_loop(0, ncA, lz_chunk, 0)

            dxi.wait()
            dxj.wait()
            for z in range(D // L):
                sz = pl.ds(z * L, L)
                xi = xrow_i[sz]
                xj = xrow_j[sz]
                xs_row[pl.ds(z * L, L)] = xi * xj
                xs_row[pl.ds(D + z * L, L)] = w * xi
                xs_row[pl.ds(2 * D + z * L, L)] = w * xj
            pltpu.sync_copy(xs_row, xs_hbm.at[_ds8((pb + k) * 4 * D, 4 * D)])
            return 0

        lax.fori_loop(0, PPT, pair_body, 0)
        scope4.__exit__(None, None, None)

    return sc_build


def _mlp_body(xs_ref, wlin_ref, blin_ref, w1_ref, b1_ref, w2_ref, b2_ref,
              out_ref):
    xs = xs_ref[...]
    h = jnp.dot(xs, w1_ref[...], preferred_element_type=jnp.float32)
    h = jnp.maximum(h + b1_ref[...], 0.0)
    out = jnp.dot(xs, wlin_ref[...], preferred_element_type=jnp.float32)
    out = out + jnp.dot(h, w2_ref[...], preferred_element_type=jnp.float32)
    out_ref[...] = out + blin_ref[...] + b2_ref[...]


def kernel(x, edge_index, tar_ei, W_lin, b_lin, W1, b1, W2, b2):
    N, D = x.shape
    E = edge_index.shape[1]
    B = tar_ei.shape[1]
    sc_build = _make_sc_builder(N, E, B, D)
    xs = sc_build(x.reshape(-1), edge_index.astype(jnp.int32).reshape(-1),
                  tar_ei.astype(jnp.int32).reshape(-1)).reshape(B, 4 * D)
    out = pl.pallas_call(
        _mlp_body,
        out_shape=jax.ShapeDtypeStruct((B, W_lin.shape[1]), jnp.float32),
    )(xs, W_lin, b_lin.reshape(1, -1), W1, b1.reshape(1, -1), W2,
      b2.reshape(1, -1))
    return out


# speedup vs baseline: 1.3311x; 1.0635x over previous
import functools

import jax
import jax.numpy as jnp
from jax import lax
from jax.experimental import pallas as pl
from jax.experimental.pallas import tpu as pltpu
from jax.experimental.pallas import tpu_sc as plsc

L = 16
NC = 2
NS = 16


def _ds8(off, size):
    return pl.ds(pl.multiple_of(off, 8), size)


def _scal(vec):
    return vec[0]


def _sget(ref, k):
    idx = jnp.full((L,), k, jnp.int32)
    return _scal(plsc.load_gather(ref, [idx], mask=jnp.full((L,), True)))


def _make_sc_builder(N, E, B, D):
    NPAD = ((N + 127) // 128) * 128
    NSLOT = 2 * B
    EPT = E // NS
    CH = 4000
    assert EPT % CH == 0 and CH % L == 0
    CAP = 2 * E + 8 * NSLOT + 256
    TRASH = CAP - 8
    RELCAP = 2 * (E // NS) + 256
    PPT = B // (NC * NS)
    assert PPT * NC * NS == B and NSLOT % L == 0 and B % L == 0

    mesh = plsc.VectorSubcoreMesh(core_axis_name="c", subcore_axis_name="s",
                                  num_cores=NC, num_subcores=NS)

    @functools.partial(
        pl.kernel,
        out_type=jax.ShapeDtypeStruct((B * 4 * D,), jnp.float32),
        mesh=mesh,
        compiler_params=pltpu.CompilerParams(needs_layout_passes=False),
        scratch_types=dict(
            T_sh=pltpu.VMEM_SHARED((NPAD,), jnp.int32),
            H_sh=pltpu.VMEM_SHARED((NS, NSLOT), jnp.int32),
            LEN_sh=pltpu.VMEM_SHARED((NSLOT,), jnp.int32),
            BASE_sh=pltpu.VMEM_SHARED((NSLOT,), jnp.int32),
            CSR_sh=pltpu.VMEM_SHARED((CAP,), jnp.int32),
            REL_sh=pltpu.VMEM_SHARED((NS * RELCAP,), jnp.int32),
            T_loc=pltpu.VMEM((NPAD,), jnp.int32),
            cnt=pltpu.VMEM((NPAD,), jnp.int32),
            hist=pltpu.VMEM((NSLOT,), jnp.int32),
            cursor=pltpu.VMEM((NSLOT,), jnp.int32),
            hrow=pltpu.VMEM((NSLOT,), jnp.int32),
            base_loc=pltpu.VMEM((NSLOT,), jnp.int32),
            len_loc=pltpu.VMEM((NSLOT,), jnp.int32),
            ubuf=pltpu.VMEM((CH,), jnp.int32),
            vbuf=pltpu.VMEM((CH,), jnp.int32),
            tar_full=pltpu.VMEM((NSLOT,), jnp.int32),
            tari_loc=pltpu.VMEM((PPT,), jnp.int32),
            tarj_loc=pltpu.VMEM((PPT,), jnp.int32),
            slots_i=pltpu.VMEM((PPT,), jnp.int32),
            slots_j=pltpu.VMEM((PPT,), jnp.int32),
            lbuf=pltpu.VMEM((128,), jnp.int32),
            rbuf=pltpu.VMEM((128,), jnp.int32),
            stage_pos=pltpu.VMEM((160,), jnp.int32),
            stage_val=pltpu.VMEM((160,), jnp.int32),
            flush_pos=pltpu.VMEM((128,), jnp.int32),
            flush_val=pltpu.VMEM((128,), jnp.int32),
            stage_wv=pltpu.VMEM((L,), jnp.int32),
            stage_wc=pltpu.VMEM((L,), jnp.int32),
            xrow_i=pltpu.VMEM((D,), jnp.float32),
            xrow_j=pltpu.VMEM((D,), jnp.float32),
            xrow_w=pltpu.VMEM((D,), jnp.float32),
            xs_row=pltpu.VMEM((4 * D,), jnp.float32),
            psem=pltpu.SemaphoreType.DMA,
        ),
    )
    def sc_build(x_hbm, ei_hbm, tar_hbm, xs_hbm, *, T_sh, H_sh, LEN_sh,
                 BASE_sh, CSR_sh, T_loc, cnt, hist, cursor, hrow, base_loc,
                 len_loc, ubuf, vbuf, tar_full, tari_loc, tarj_loc, slots_i,
                 slots_j, lbuf, rbuf, REL_sh, stage_pos, stage_val, flush_pos,
                 flush_val, stage_wv, stage_wc, xrow_i, xrow_j, xrow_w,
                 xs_row, psem):
        cid = lax.axis_index("c")
        sid = lax.axis_index("s")
        lane = lax.iota(jnp.int32, L)
        zeros16 = jnp.zeros((L,), jnp.int32)

        scope0 = jax.named_scope("sc_p0_slot_table")
        scope0.__enter__()
        @pl.when(sid == 0)
        def _():
            neg1 = jnp.full((L,), -1, jnp.int32)

            def t_init(i, _):
                T_loc[pl.ds(i * L, L)] = neg1
                return 0

            lax.fori_loop(0, NPAD // L, t_init, 0)
            pltpu.sync_copy(tar_hbm, tar_full)

            def t_scat(i, _):
                nodes = tar_full[pl.ds(i * L, L)]
                plsc.store_scatter(T_loc, [nodes], lane + i * L,
                                   mask=jnp.full((L,), True))
                return 0

            lax.fori_loop(0, NSLOT // L, t_scat, 0)
            pltpu.sync_copy(T_loc, T_sh)

        plsc.subcore_barrier()
        scope0.__exit__(None, None, None)

        scope1 = jax.named_scope("sc_p1_histogram")
        scope1.__enter__()
        pltpu.sync_copy(T_sh, T_loc)

        def z_hist(i, _):
            hist[pl.ds(i * L, L)] = zeros16
            return 0

        lax.fori_loop(0, NSLOT // L, z_hist, 0)

        def z_cnt(i, _):
            cnt[pl.ds(i * L, L)] = zeros16
            return 0

        lax.fori_loop(0, NPAD // L, z_cnt, 0)

        ebase = sid * EPT
        rbase = sid * RELCAP

        def hist_chunk(c, fill):
            pltpu.sync_copy(ei_hbm.at[_ds8(ebase + c * CH, CH)], ubuf)
            pltpu.sync_copy(ei_hbm.at[_ds8(E + ebase + c * CH, CH)], vbuf)

            def rel_flush(carry):
                fill, nfl = carry
                pltpu.sync_copy(stage_pos.at[pl.ds(0, 128)],
                                REL_sh.at[_ds8(rbase + nfl * 128, 128)])
                stage_pos[pl.ds(0, L)] = stage_pos[pl.ds(128, L)]
                return fill - 128, nfl + 1

            def hist_vec(i, carry):
                fill, nfl = carry
                u = ubuf[pl.ds(i * L, L)]
                v = vbuf[pl.ds(i * L, L)]
                for a, bb in ((u, v), (v, u)):
                    r = plsc.load_gather(T_loc, [a], mask=jnp.full((L,), True))
                    m = r >= 0
                    rs = jnp.where(m, r, 0)
                    cntv, lastm = plsc.scan_count(rs, m)
                    plsc.addupdate_scatter(hist, [rs], cntv,
                                           mask=jnp.logical_and(m, lastm))
                    pk = (rs << 14) | bb
                    plsc.store_compressed(stage_pos.at[pl.ds(fill, L)], pk,
                                          mask=m)
                    fill = fill + _scal(plsc.all_reduce_population_count(m))
                    fill, nfl = lax.cond(fill >= 128, rel_flush,
                                         lambda c: c, (fill, nfl))
                return fill, nfl

            return lax.fori_loop(0, CH // L, hist_vec, fill)

        fill, nfl = lax.fori_loop(0, EPT // CH, hist_chunk,
                                  (jnp.int32(0), jnp.int32(0)))
        pltpu.sync_copy(stage_pos.at[pl.ds(0, 128)],
                        REL_sh.at[_ds8(rbase + nfl * 128, 128)])
        nrel = nfl * 128 + fill
        pltpu.sync_copy(hist, H_sh.at[sid])
        plsc.subcore_barrier()
        scope1.__exit__(None, None, None)

        scope2 = jax.named_scope("sc_p2_offsets")
        scope2.__enter__()
        @pl.when(sid == 0)
        def _():
            def z_len(i, _):
                len_loc[pl.ds(i * L, L)] = zeros16
                return 0

            lax.fori_loop(0, NSLOT // L, z_len, 0)

            def acc_tile(t, _):
                pltpu.sync_copy(H_sh.at[t], hrow)

                def acc_vec(i, _):
                    s = pl.ds(i * L, L)
                    len_loc[s] = len_loc[s] + hrow[s]
                    return 0

                lax.fori_loop(0, NSLOT // L, acc_vec, 0)
                return 0

            lax.fori_loop(0, NS, acc_tile, 0)
            pltpu.sync_copy(len_loc, LEN_sh)

            def base_vec(i, carry):
                s = pl.ds(i * L, L)
                lv = len_loc[s]
                lp = (lv + 7) & jnp.int32(~7)
                cs = plsc.cumsum(lp)
                base_loc[s] = carry + cs - lp
                return carry + cs[L - 1]

            lax.fori_loop(0, NSLOT // L, base_vec, jnp.int32(0))
            pltpu.sync_copy(base_loc, BASE_sh)

        plsc.subcore_barrier()
        scope2.__exit__(None, None, None)

        scope3 = jax.named_scope("sc_p3_placement")
        scope3.__enter__()
        pltpu.sync_copy(BASE_sh, cursor)

        def pref_tile(t, _):
            pltpu.sync_copy(H_sh.at[t], hrow)

            def pref_vec(i, _):
                s = pl.ds(i * L, L)
                cursor[s] = cursor[s] + hrow[s]
                return 0

            lax.fori_loop(0, NSLOT // L, pref_vec, 0)
            return 0

        lax.fori_loop(0, sid, pref_tile, 0)

        def do_flush(f):
            for k in range(8):
                s = pl.ds(k * L, L)
                flush_pos[s] = stage_pos[s]
                flush_val[s] = stage_val[s]
            pltpu.sync_copy(flush_val, CSR_sh.at[flush_pos])
            stage_pos[pl.ds(0, L)] = stage_pos[pl.ds(128, L)]
            stage_val[pl.ds(0, L)] = stage_val[pl.ds(128, L)]
            return f - 128

        def place_chunk(cc, fill):
            pltpu.sync_copy(REL_sh.at[_ds8(rbase + cc * 128, 128)], rbuf)

            def place_vec(q, fill):
                pk = rbuf[pl.ds(q * L, L)]
                m = (cc * 128 + q * L + lane) < nrel
                rs = jnp.where(m, pk >> 14, 0)
                vv = pk & 16383
                cntv, lastm = plsc.scan_count(rs, m)
                before = plsc.load_gather(cursor, [rs], mask=m)
                pos = before + cntv - 1
                plsc.addupdate_scatter(cursor, [rs], cntv,
                                       mask=jnp.logical_and(m, lastm))
                plsc.store_compressed(stage_pos.at[pl.ds(fill, L)], pos,
                                      mask=m)
                plsc.store_compressed(stage_val.at[pl.ds(fill, L)], vv,
                                      mask=m)
                fill = fill + _scal(plsc.all_reduce_population_count(m))
                return lax.cond(fill >= 128, do_flush, lambda f: f, fill)

            return lax.fori_loop(0, 8, place_vec, fill)

        fill = lax.fori_loop(0, (nrel + 127) // 128, place_chunk,
                             jnp.int32(0))
        for k in range(8):
            s = pl.ds(k * L, L)
            g = lane + k * L
            flush_pos[s] = jnp.where(g < fill, stage_pos[s], TRASH)
            flush_val[s] = stage_val[s]
        pltpu.sync_copy(flush_val, CSR_sh.at[flush_pos])
        plsc.subcore_barrier()
        scope3.__exit__(None, None, None)

        scope4 = jax.named_scope("sc_p4_pairs")
        scope4.__enter__()
        pltpu.sync_copy(BASE_sh, base_loc)
        pltpu.sync_copy(LEN_sh, len_loc)
        pb = cid * (NS * PPT) + sid * PPT
        pltpu.sync_copy(tar_hbm.at[_ds8(pb, PPT)], tari_loc)
        pltpu.sync_copy(tar_hbm.at[_ds8(B + pb, PPT)], tarj_loc)

        for k in range(PPT // L):
            s = pl.ds(k * L, L)
            slots_i[s] = plsc.load_gather(T_loc, [tari_loc[s]],
                                          mask=jnp.full((L,), True))
            slots_j[s] = plsc.load_gather(T_loc, [tarj_loc[s]],
                                          mask=jnp.full((L,), True))

        zf16 = jnp.zeros((L,), jnp.float32)

        def pair_body(k, _):
            ib = _sget(tari_loc, k)
            jb = _sget(tarj_loc, k)
            ri = _sget(slots_i, k)
            rj = _sget(slots_j, k)
            baseA = _sget(base_loc, ri)
            lenA = _sget(len_loc, ri)
            baseB = _sget(base_loc, rj)
            lenB = _sget(len_loc, rj)
            ncA = (lenA + 127) // 128
            nvA = (lenA + L - 1) // L
            nvB = (lenB + L - 1) // L

            dxi = pltpu.async_copy(x_hbm.at[_ds8(ib * D, D)], xrow_i, psem)
            dxj = pltpu.async_copy(x_hbm.at[_ds8(jb * D, D)], xrow_j, psem)

            for z in range(D // L):
                xs_row[pl.ds(3 * D + z * L, L)] = zf16

            def la_chunk(cc, _):
                pltpu.sync_copy(CSR_sh.at[_ds8(baseA + cc * 128, 128)], lbuf)

                def la_vec(q, _):
                    vv = lbuf[pl.ds(q * L, L)]
                    m = (cc * 128 + q * L + lane) < lenA
                    vs = jnp.where(m, vv, 0)
                    cntv, lastm = plsc.scan_count(vs, m)
                    plsc.addupdate_scatter(cnt, [vs], cntv,
                                           mask=jnp.logical_and(m, lastm))
                    return 0

                lax.fori_loop(0, jnp.minimum(8, nvA - cc * 8), la_vec, 0)
                return 0

            lax.fori_loop(0, ncA, la_chunk, 0)

            w = _sget(cnt, jb).astype(jnp.float32)

            def lb_chunk(cc, _):
                pltpu.sync_copy(CSR_sh.at[_ds8(baseB + cc * 128, 128)], rbuf)

                def lb_vec(q, _):
                    vv = rbuf[pl.ds(q * L, L)]
                    m = (cc * 128 + q * L + lane) < lenB
                    vs = jnp.where(m, vv, 0)
                    cv = plsc.load_gather(cnt, [vs], mask=m)
                    wm = jnp.logical_and(m, cv > 0)
                    nn = _scal(plsc.all_reduce_population_count(wm))

                    @pl.when(nn > 0)
                    def _():
                        plsc.store_compressed(stage_wv.at[pl.ds(0, L)], vs,
                                              mask=wm)
                        plsc.store_compressed(stage_wc.at[pl.ds(0, L)], cv,
                                              mask=wm)

                        def work(t, _):
                            wv = _sget(stage_wv, t)
                            wc = _sget(stage_wc, t).astype(jnp.float32)
                            pltpu.sync_copy(x_hbm.at[_ds8(wv * D, D)],
                                            xrow_w)
                            for z in range(D // L):
                                sl = pl.ds(3 * D + z * L, L)
                                xs_row[sl] = (xs_row[sl]
                                              + wc * xrow_w[pl.ds(z * L, L)])
                            return 0

                        lax.fori_loop(0, nn, work, 0)

                    return 0

                lax.fori_loop(0, jnp.minimum(8, nvB - cc * 8), lb_vec, 0)
                return 0

            lax.fori_loop(0, (lenB + 127) // 128, lb_chunk, 0)

            def lz_chunk(cc, _):
                @pl.when(ncA > 1)
                def _():
                    pltpu.sync_copy(CSR_sh.at[_ds8(baseA + cc * 128, 128)],
                                    lbuf)

                def lz_vec(q, _):
                    vv = lbuf[pl.ds(q * L, L)]
                    m = (cc * 128 + q * L + lane) < lenA
                    vs = jnp.where(m, vv, 0)
                    plsc.store_scatter(cnt, [vs], zeros16, mask=m)
                    return 0

                lax.fori_loop(0, jnp.minimum(8, nvA - cc * 8), lz_vec, 0)
                return 0

            lax.fori_loop(0, ncA, lz_chunk, 0)

            dxi.wait()
            dxj.wait()
            for z in range(D // L):
                sz = pl.ds(z * L, L)
                xi = xrow_i[sz]
                xj = xrow_j[sz]
                xs_row[pl.ds(z * L, L)] = xi * xj
                xs_row[pl.ds(D + z * L, L)] = w * xi
                xs_row[pl.ds(2 * D + z * L, L)] = w * xj
            pltpu.sync_copy(xs_row, xs_hbm.at[_ds8((pb + k) * 4 * D, 4 * D)])
            return 0

        lax.fori_loop(0, PPT, pair_body, 0)
        scope4.__exit__(None, None, None)

    return sc_build


def _mlp_body(xs_ref, wlin_ref, blin_ref, w1_ref, b1_ref, w2_ref, b2_ref,
              out_ref):
    xs = xs_ref[...]
    h = jnp.dot(xs, w1_ref[...], preferred_element_type=jnp.float32)
    h = jnp.maximum(h + b1_ref[...], 0.0)
    out = jnp.dot(xs, wlin_ref[...], preferred_element_type=jnp.float32)
    out = out + jnp.dot(h, w2_ref[...], preferred_element_type=jnp.float32)
    out_ref[...] = out + blin_ref[...] + b2_ref[...]


def kernel(x, edge_index, tar_ei, W_lin, b_lin, W1, b1, W2, b2):
    N, D = x.shape
    E = edge_index.shape[1]
    B = tar_ei.shape[1]
    sc_build = _make_sc_builder(N, E, B, D)
    xs = sc_build(x.reshape(-1), edge_index.astype(jnp.int32).reshape(-1),
                  tar_ei.astype(jnp.int32).reshape(-1)).reshape(B, 4 * D)
    out = pl.pallas_call(
        _mlp_body,
        out_shape=jax.ShapeDtypeStruct((B, W_lin.shape[1]), jnp.float32),
    )(xs, W_lin, b_lin.reshape(1, -1), W1, b1.reshape(1, -1), W2,
      b2.reshape(1, -1))
    return out


# speedup vs baseline: 1.4197x; 1.0666x over previous
import functools

import jax
import jax.numpy as jnp
from jax import lax
from jax.experimental import pallas as pl
from jax.experimental.pallas import tpu as pltpu
from jax.experimental.pallas import tpu_sc as plsc

L = 16
NC = 2
NS = 16


def _ds8(off, size):
    return pl.ds(pl.multiple_of(off, 8), size)


def _scal(vec):
    return vec[0]


def _sget(ref, k):
    idx = jnp.full((L,), k, jnp.int32)
    return _scal(plsc.load_gather(ref, [idx], mask=jnp.full((L,), True)))


def _make_sc_builder(N, E, B, D):
    NPAD = ((N + 127) // 128) * 128
    NSLOT = 2 * B
    EPT = E // NS
    CH = 2000
    assert EPT % CH == 0 and CH % L == 0
    CAP = 2 * E + 8 * NSLOT + 256
    TRASH = CAP - 8
    RELCAP = 2 * (E // NS) + 256
    PPT = B // (NC * NS)
    assert PPT * NC * NS == B and NSLOT % L == 0 and B % L == 0

    mesh = plsc.VectorSubcoreMesh(core_axis_name="c", subcore_axis_name="s",
                                  num_cores=NC, num_subcores=NS)

    @functools.partial(
        pl.kernel,
        out_type=jax.ShapeDtypeStruct((B * 4 * D,), jnp.float32),
        mesh=mesh,
        compiler_params=pltpu.CompilerParams(needs_layout_passes=False),
        scratch_types=dict(
            T_sh=pltpu.VMEM_SHARED((NPAD,), jnp.int32),
            H_sh=pltpu.VMEM_SHARED((NS, NSLOT), jnp.int32),
            LEN_sh=pltpu.VMEM_SHARED((NSLOT,), jnp.int32),
            BASE_sh=pltpu.VMEM_SHARED((NSLOT,), jnp.int32),
            CSR_sh=pltpu.VMEM_SHARED((CAP,), jnp.int32),
            REL_sh=pltpu.VMEM_SHARED((NS * RELCAP,), jnp.int32),
            T_loc=pltpu.VMEM((NPAD,), jnp.int32),
            cnt=pltpu.VMEM((NPAD,), jnp.int32),
            hist=pltpu.VMEM((NSLOT,), jnp.int32),
            cursor=pltpu.VMEM((NSLOT,), jnp.int32),
            hrow=pltpu.VMEM((NSLOT,), jnp.int32),
            base_loc=pltpu.VMEM((NSLOT,), jnp.int32),
            len_loc=pltpu.VMEM((NSLOT,), jnp.int32),
            ubuf=pltpu.VMEM((CH,), jnp.int32),
            vbuf=pltpu.VMEM((CH,), jnp.int32),
            ubuf2=pltpu.VMEM((CH,), jnp.int32),
            vbuf2=pltpu.VMEM((CH,), jnp.int32),
            tar_full=pltpu.VMEM((NSLOT,), jnp.int32),
            tari_loc=pltpu.VMEM((PPT,), jnp.int32),
            tarj_loc=pltpu.VMEM((PPT,), jnp.int32),
            slots_i=pltpu.VMEM((PPT,), jnp.int32),
            slots_j=pltpu.VMEM((PPT,), jnp.int32),
            lbuf=pltpu.VMEM((128,), jnp.int32),
            rbuf=pltpu.VMEM((128,), jnp.int32),
            stage_pos=pltpu.VMEM((160,), jnp.int32),
            stage_val=pltpu.VMEM((160,), jnp.int32),
            flush_pos=pltpu.VMEM((128,), jnp.int32),
            flush_val=pltpu.VMEM((128,), jnp.int32),
            stage_wv=pltpu.VMEM((L,), jnp.int32),
            stage_wc=pltpu.VMEM((L,), jnp.int32),
            xrow_i=pltpu.VMEM((D,), jnp.float32),
            xrow_j=pltpu.VMEM((D,), jnp.float32),
            xrow_w=pltpu.VMEM((D,), jnp.float32),
            xs_row=pltpu.VMEM((4 * D,), jnp.float32),
            psem=pltpu.SemaphoreType.DMA,
        ),
    )
    def sc_build(x_hbm, ei_hbm, tar_hbm, xs_hbm, *, T_sh, H_sh, LEN_sh,
                 BASE_sh, CSR_sh, T_loc, cnt, hist, cursor, hrow, base_loc,
                 len_loc, ubuf, vbuf, ubuf2, vbuf2, tar_full, tari_loc,
                 tarj_loc, slots_i,
                 slots_j, lbuf, rbuf, REL_sh, stage_pos, stage_val, flush_pos,
                 flush_val, stage_wv, stage_wc, xrow_i, xrow_j, xrow_w,
                 xs_row, psem):
        cid = lax.axis_index("c")
        sid = lax.axis_index("s")
        lane = lax.iota(jnp.int32, L)
        zeros16 = jnp.zeros((L,), jnp.int32)

        scope0 = jax.named_scope("sc_p0_slot_table")
        scope0.__enter__()
        @pl.when(sid == 0)
        def _():
            neg1 = jnp.full((L,), -1, jnp.int32)

            def t_init(i, _):
                T_loc[pl.ds(i * L, L)] = neg1
                return 0

            lax.fori_loop(0, NPAD // L, t_init, 0)
            pltpu.sync_copy(tar_hbm, tar_full)

            def t_scat(i, _):
                nodes = tar_full[pl.ds(i * L, L)]
                plsc.store_scatter(T_loc, [nodes], lane + i * L,
                                   mask=jnp.full((L,), True))
                return 0

            lax.fori_loop(0, NSLOT // L, t_scat, 0)
            pltpu.sync_copy(T_loc, T_sh)

        plsc.subcore_barrier()
        scope0.__exit__(None, None, None)

        scope1 = jax.named_scope("sc_p1_histogram")
        scope1.__enter__()
        pltpu.sync_copy(T_sh, T_loc)

        def z_hist(i, _):
            hist[pl.ds(i * L, L)] = zeros16
            return 0

        lax.fori_loop(0, NSLOT // L, z_hist, 0)

        def z_cnt(i, _):
            cnt[pl.ds(i * L, L)] = zeros16
            return 0

        lax.fori_loop(0, NPAD // L, z_cnt, 0)

        ebase = sid * EPT
        rbase = sid * RELCAP

        def rel_flush(carry):
            fill, nfl = carry
            pltpu.sync_copy(stage_pos.at[pl.ds(0, 128)],
                            REL_sh.at[_ds8(rbase + nfl * 128, 128)])
            stage_pos[pl.ds(0, L)] = stage_pos[pl.ds(128, L)]
            return fill - 128, nfl + 1

        NCHUNK = EPT // CH
        bufs = [(ubuf, vbuf), (ubuf2, vbuf2)]
        pltpu.sync_copy(ei_hbm.at[_ds8(ebase, CH)], ubuf)
        pltpu.sync_copy(ei_hbm.at[_ds8(E + ebase, CH)], vbuf)
        carry = (jnp.int32(0), jnp.int32(0))
        for c in range(NCHUNK):
            ub, vb = bufs[c % 2]
            un, vn = bufs[(c + 1) % 2]
            if c + 1 < NCHUNK:
                dn1 = pltpu.async_copy(
                    ei_hbm.at[_ds8(ebase + (c + 1) * CH, CH)], un, psem)
                dn2 = pltpu.async_copy(
                    ei_hbm.at[_ds8(E + ebase + (c + 1) * CH, CH)], vn, psem)

            def hist_vec(i, carry, ub=ub, vb=vb):
                fill, nfl = carry
                u = ub[pl.ds(i * L, L)]
                v = vb[pl.ds(i * L, L)]
                for a, bb in ((u, v), (v, u)):
                    r = plsc.load_gather(T_loc, [a], mask=jnp.full((L,), True))
                    m = r >= 0
                    rs = jnp.where(m, r, 0)
                    cntv, lastm = plsc.scan_count(rs, m)
                    plsc.addupdate_scatter(hist, [rs], cntv,
                                           mask=jnp.logical_and(m, lastm))
                    pk = (rs << 14) | bb
                    plsc.store_compressed(stage_pos.at[pl.ds(fill, L)], pk,
                                          mask=m)
                    fill = fill + _scal(plsc.all_reduce_population_count(m))
                fill, nfl = lax.cond(fill >= 128, rel_flush,
                                     lambda c: c, (fill, nfl))
                return fill, nfl

            carry = lax.fori_loop(0, CH // L, hist_vec, carry)
            if c + 1 < NCHUNK:
                dn1.wait()
                dn2.wait()
        fill, nfl = carry
        pltpu.sync_copy(stage_pos.at[pl.ds(0, 128)],
                        REL_sh.at[_ds8(rbase + nfl * 128, 128)])
        nrel = nfl * 128 + fill
        pltpu.sync_copy(hist, H_sh.at[sid])
        plsc.subcore_barrier()
        scope1.__exit__(None, None, None)

        scope2 = jax.named_scope("sc_p2_offsets")
        scope2.__enter__()
        @pl.when(sid == 0)
        def _():
            def z_len(i, _):
                len_loc[pl.ds(i * L, L)] = zeros16
                return 0

            lax.fori_loop(0, NSLOT // L, z_len, 0)

            def acc_tile(t, _):
                pltpu.sync_copy(H_sh.at[t], hrow)

                def acc_vec(i, _):
                    s = pl.ds(i * L, L)
                    len_loc[s] = len_loc[s] + hrow[s]
                    return 0

                lax.fori_loop(0, NSLOT // L, acc_vec, 0)
                return 0

            lax.fori_loop(0, NS, acc_tile, 0)
            pltpu.sync_copy(len_loc, LEN_sh)

            def base_vec(i, carry):
                s = pl.ds(i * L, L)
                lv = len_loc[s]
                lp = (lv + 7) & jnp.int32(~7)
                cs = plsc.cumsum(lp)
                base_loc[s] = carry + cs - lp
                return carry + cs[L - 1]

            lax.fori_loop(0, NSLOT // L, base_vec, jnp.int32(0))
            pltpu.sync_copy(base_loc, BASE_sh)

        plsc.subcore_barrier()
        scope2.__exit__(None, None, None)

        scope3 = jax.named_scope("sc_p3_placement")
        scope3.__enter__()
        pltpu.sync_copy(BASE_sh, cursor)

        def pref_tile(t, _):
            pltpu.sync_copy(H_sh.at[t], hrow)

            def pref_vec(i, _):
                s = pl.ds(i * L, L)
                cursor[s] = cursor[s] + hrow[s]
                return 0

            lax.fori_loop(0, NSLOT // L, pref_vec, 0)
            return 0

        lax.fori_loop(0, sid, pref_tile, 0)

        def do_flush(f):
            for k in range(8):
                s = pl.ds(k * L, L)
                flush_pos[s] = stage_pos[s]
                flush_val[s] = stage_val[s]
            pltpu.sync_copy(flush_val, CSR_sh.at[flush_pos])
            stage_pos[pl.ds(0, L)] = stage_pos[pl.ds(128, L)]
            stage_val[pl.ds(0, L)] = stage_val[pl.ds(128, L)]
            return f - 128

        def place_chunk(cc, fill):
            pltpu.sync_copy(REL_sh.at[_ds8(rbase + cc * 128, 128)], rbuf)

            def place_vec(q, fill):
                pk = rbuf[pl.ds(q * L, L)]
                m = (cc * 128 + q * L + lane) < nrel
                rs = jnp.where(m, pk >> 14, 0)
                vv = pk & 16383
                cntv, lastm = plsc.scan_count(rs, m)
                before = plsc.load_gather(cursor, [rs], mask=m)
                pos = before + cntv - 1
                plsc.addupdate_scatter(cursor, [rs], cntv,
                                       mask=jnp.logical_and(m, lastm))
                plsc.store_compressed(stage_pos.at[pl.ds(fill, L)], pos,
                                      mask=m)
                plsc.store_compressed(stage_val.at[pl.ds(fill, L)], vv,
                                      mask=m)
                fill = fill + _scal(plsc.all_reduce_population_count(m))
                return lax.cond(fill >= 128, do_flush, lambda f: f, fill)

            return lax.fori_loop(0, 8, place_vec, fill)

        fill = lax.fori_loop(0, (nrel + 127) // 128, place_chunk,
                             jnp.int32(0))
        for k in range(8):
            s = pl.ds(k * L, L)
            g = lane + k * L
            flush_pos[s] = jnp.where(g < fill, stage_pos[s], TRASH)
            flush_val[s] = stage_val[s]
        pltpu.sync_copy(flush_val, CSR_sh.at[flush_pos])
        plsc.subcore_barrier()
        scope3.__exit__(None, None, None)

        scope4 = jax.named_scope("sc_p4_pairs")
        scope4.__enter__()
        pltpu.sync_copy(BASE_sh, base_loc)
        pltpu.sync_copy(LEN_sh, len_loc)
        pb = cid * (NS * PPT) + sid * PPT
        pltpu.sync_copy(tar_hbm.at[_ds8(pb, PPT)], tari_loc)
        pltpu.sync_copy(tar_hbm.at[_ds8(B + pb, PPT)], tarj_loc)

        for k in range(PPT // L):
            s = pl.ds(k * L, L)
            slots_i[s] = plsc.load_gather(T_loc, [tari_loc[s]],
                                          mask=jnp.full((L,), True))
            slots_j[s] = plsc.load_gather(T_loc, [tarj_loc[s]],
                                          mask=jnp.full((L,), True))

        zf16 = jnp.zeros((L,), jnp.float32)

        def pair_body(k, _):
            ib = _sget(tari_loc, k)
            jb = _sget(tarj_loc, k)
            ri = _sget(slots_i, k)
            rj = _sget(slots_j, k)
            baseA = _sget(base_loc, ri)
            lenA = _sget(len_loc, ri)
            baseB = _sget(base_loc, rj)
            lenB = _sget(len_loc, rj)
            ncA = (lenA + 127) // 128
            nvA = (lenA + L - 1) // L
            nvB = (lenB + L - 1) // L

            dxi = pltpu.async_copy(x_hbm.at[_ds8(ib * D, D)], xrow_i, psem)
            dxj = pltpu.async_copy(x_hbm.at[_ds8(jb * D, D)], xrow_j, psem)

            for z in range(D // L):
                xs_row[pl.ds(3 * D + z * L, L)] = zf16

            def la_chunk(cc, _):
                pltpu.sync_copy(CSR_sh.at[_ds8(baseA + cc * 128, 128)], lbuf)

                def la_vec(q, _):
                    vv = lbuf[pl.ds(q * L, L)]
                    m = (cc * 128 + q * L + lane) < lenA
                    vs = jnp.where(m, vv, 0)
                    cntv, lastm = plsc.scan_count(vs, m)
                    plsc.addupdate_scatter(cnt, [vs], cntv,
                                           mask=jnp.logical_and(m, lastm))
                    return 0

                lax.fori_loop(0, jnp.minimum(8, nvA - cc * 8), la_vec, 0)
                return 0

            lax.fori_loop(0, ncA, la_chunk, 0)

            w = _sget(cnt, jb).astype(jnp.float32)

            def lb_chunk(cc, _):
                pltpu.sync_copy(CSR_sh.at[_ds8(baseB + cc * 128, 128)], rbuf)

                def lb_vec(q, _):
                    vv = rbuf[pl.ds(q * L, L)]
                    m = (cc * 128 + q * L + lane) < lenB
                    vs = jnp.where(m, vv, 0)
                    cv = plsc.load_gather(cnt, [vs], mask=m)
                    wm = jnp.logical_and(m, cv > 0)
                    nn = _scal(plsc.all_reduce_population_count(wm))

                    @pl.when(nn > 0)
                    def _():
                        plsc.store_compressed(stage_wv.at[pl.ds(0, L)], vs,
                                              mask=wm)
                        plsc.store_compressed(stage_wc.at[pl.ds(0, L)], cv,
                                              mask=wm)

                        def work(t, _):
                            wv = _sget(stage_wv, t)
                            wc = _sget(stage_wc, t).astype(jnp.float32)
                            pltpu.sync_copy(x_hbm.at[_ds8(wv * D, D)],
                                            xrow_w)
                            for z in range(D // L):
                                sl = pl.ds(3 * D + z * L, L)
                                xs_row[sl] = (xs_row[sl]
                                              + wc * xrow_w[pl.ds(z * L, L)])
                            return 0

                        lax.fori_loop(0, nn, work, 0)

                    return 0

                lax.fori_loop(0, jnp.minimum(8, nvB - cc * 8), lb_vec, 0)
                return 0

            lax.fori_loop(0, (lenB + 127) // 128, lb_chunk, 0)

            def lz_chunk(cc, _):
                @pl.when(ncA > 1)
                def _():
                    pltpu.sync_copy(CSR_sh.at[_ds8(baseA + cc * 128, 128)],
                                    lbuf)

                def lz_vec(q, _):
                    vv = lbuf[pl.ds(q * L, L)]
                    m = (cc * 128 + q * L + lane) < lenA
                    vs = jnp.where(m, vv, 0)
                    plsc.store_scatter(cnt, [vs], zeros16, mask=m)
                    return 0

                lax.fori_loop(0, jnp.minimum(8, nvA - cc * 8), lz_vec, 0)
                return 0

            lax.fori_loop(0, ncA, lz_chunk, 0)

            dxi.wait()
            dxj.wait()
            for z in range(D // L):
                sz = pl.ds(z * L, L)
                xi = xrow_i[sz]
                xj = xrow_j[sz]
                xs_row[pl.ds(z * L, L)] = xi * xj
                xs_row[pl.ds(D + z * L, L)] = w * xi
                xs_row[pl.ds(2 * D + z * L, L)] = w * xj
            pltpu.sync_copy(xs_row, xs_hbm.at[_ds8((pb + k) * 4 * D, 4 * D)])
            return 0

        lax.fori_loop(0, PPT, pair_body, 0)
        scope4.__exit__(None, None, None)

    return sc_build


def _mlp_body(xs_ref, wlin_ref, blin_ref, w1_ref, b1_ref, w2_ref, b2_ref,
              out_ref):
    xs = xs_ref[...]
    h = jnp.dot(xs, w1_ref[...], preferred_element_type=jnp.float32)
    h = jnp.maximum(h + b1_ref[...], 0.0)
    out = jnp.dot(xs, wlin_ref[...], preferred_element_type=jnp.float32)
    out = out + jnp.dot(h, w2_ref[...], preferred_element_type=jnp.float32)
    out_ref[...] = out + blin_ref[...] + b2_ref[...]


def kernel(x, edge_index, tar_ei, W_lin, b_lin, W1, b1, W2, b2):
    N, D = x.shape
    E = edge_index.shape[1]
    B = tar_ei.shape[1]
    sc_build = _make_sc_builder(N, E, B, D)
    xs = sc_build(x.reshape(-1), edge_index.astype(jnp.int32).reshape(-1),
                  tar_ei.astype(jnp.int32).reshape(-1)).reshape(B, 4 * D)
    out = pl.pallas_call(
        _mlp_body,
        out_shape=jax.ShapeDtypeStruct((B, W_lin.shape[1]), jnp.float32),
    )(xs, W_lin, b_lin.reshape(1, -1), W1, b1.reshape(1, -1), W2,
      b2.reshape(1, -1))
    return out
